# Initial kernel scaffold; baseline (speedup 1.0000x reference)
import jax
import jax.numpy as jnp
from jax.experimental import pallas as pl


def kernel(indices, vectors):
    raise NotImplementedError("write your pallas kernel here")


# speedup vs baseline: 1.8439x; 1.8439x over previous
import functools

import jax
import jax.numpy as jnp
from jax import lax
from jax.experimental import pallas as pl
from jax.experimental.pallas import tpu as pltpu
from jax.experimental.pallas import tpu_sc as plsc


def _gather_kernel(N, V, D, n_workers, chunk):
    n_per_w = N // n_workers
    n_chunks = n_per_w // chunk
    mesh = plsc.VectorSubcoreMesh(core_axis_name="c", subcore_axis_name="s")

    @functools.partial(
        pl.kernel,
        mesh=mesh,
        out_type=jax.ShapeDtypeStruct((N, D), jnp.float32),
        scratch_types=[
            pltpu.VMEM((chunk,), jnp.int32),
            pltpu.VMEM((chunk, D), jnp.float32),
            pltpu.SemaphoreType.DMA,
        ],
        compiler_params=pltpu.CompilerParams(use_tc_tiling_on_sc=False),
    )
    def body(idx_hbm, table_hbm, out_hbm, idx_v, rows_v, sem):
        nc = plsc.get_sparse_core_info().num_cores
        wid = lax.axis_index("s") * nc + lax.axis_index("c")
        base = wid * n_per_w

        def step(i, carry):
            off = base + i * chunk
            pltpu.sync_copy(idx_hbm.at[pl.ds(off, chunk)], idx_v)
            pltpu.async_copy(table_hbm.at[idx_v], rows_v, sem).wait()
            pltpu.sync_copy(rows_v, out_hbm.at[pl.ds(off, chunk)])
            return carry

        lax.fori_loop(0, n_chunks, step, 0)

    return body


def kernel(indices, vectors):
    B, S = indices.shape
    V, D = vectors.shape
    N = B * S
    info = plsc.get_sparse_core_info()
    n_workers = info.num_cores * info.num_subcores
    chunk = 1024
    flat_idx = indices.reshape(N).astype(jnp.int32)
    out = _gather_kernel(N, V, D, n_workers, chunk)(flat_idx, vectors)
    return out.reshape(B, S, D)


# speedup vs baseline: 1.8762x; 1.0175x over previous
import functools

import jax
import jax.numpy as jnp
from jax import lax
from jax.experimental import pallas as pl
from jax.experimental.pallas import tpu as pltpu
from jax.experimental.pallas import tpu_sc as plsc

_NBUF = 4
_LOOKAHEAD = 2


def _gather_kernel(N, V, D, n_workers, chunk):
    n_per_w = N // n_workers
    n_chunks = n_per_w // chunk
    n_groups = n_chunks // _NBUF
    mesh = plsc.VectorSubcoreMesh(core_axis_name="c", subcore_axis_name="s")

    @functools.partial(
        pl.kernel,
        mesh=mesh,
        out_type=jax.ShapeDtypeStruct((N, D), jnp.float32),
        scratch_types=[
            pltpu.VMEM((n_per_w,), jnp.int32),
            pltpu.VMEM((_NBUF, chunk, D), jnp.float32),
            pltpu.SemaphoreType.DMA((_NBUF,)),
            pltpu.SemaphoreType.DMA((_NBUF,)),
        ],
        compiler_params=pltpu.CompilerParams(use_tc_tiling_on_sc=False),
    )
    def body(idx_hbm, table_hbm, out_hbm, idx_v, rows_v, g_sem, o_sem):
        nc = plsc.get_sparse_core_info().num_cores
        wid = lax.axis_index("s") * nc + lax.axis_index("c")
        base = wid * n_per_w
        pltpu.sync_copy(idx_hbm.at[pl.ds(base, n_per_w)], idx_v)

        def start_gather(i, slot):
            pltpu.async_copy(
                table_hbm.at[idx_v.at[pl.ds(i * chunk, chunk)]],
                rows_v.at[slot],
                g_sem.at[slot],
            )

        def wait_gather(i, slot):
            pltpu.make_async_copy(
                table_hbm.at[idx_v.at[pl.ds(i * chunk, chunk)]],
                rows_v.at[slot],
                g_sem.at[slot],
            ).wait()

        def start_writeout(i, slot):
            pltpu.async_copy(
                rows_v.at[slot],
                out_hbm.at[pl.ds(base + i * chunk, chunk)],
                o_sem.at[slot],
            )

        def wait_writeout(i, slot):
            pltpu.make_async_copy(
                rows_v.at[slot],
                out_hbm.at[pl.ds(base + i * chunk, chunk)],
                o_sem.at[slot],
            ).wait()

        for i in range(_LOOKAHEAD):
            start_gather(i, i % _NBUF)

        def group(g, carry):
            for b in range(_NBUF):
                i = g * _NBUF + b
                j_slot = (b + _LOOKAHEAD) % _NBUF
                prev = i + _LOOKAHEAD - _NBUF

                @pl.when(prev >= 0)
                def _():
                    wait_writeout(prev, j_slot)

                @pl.when(i + _LOOKAHEAD < n_chunks)
                def _():
                    start_gather(i + _LOOKAHEAD, j_slot)

                wait_gather(i, b)
                start_writeout(i, b)
            return carry

        lax.fori_loop(0, n_groups, group, 0)

        for k in range(_NBUF - _LOOKAHEAD):
            i = n_chunks - (_NBUF - _LOOKAHEAD) + k
            wait_writeout(i, i % _NBUF)

    return body


def kernel(indices, vectors):
    B, S = indices.shape
    V, D = vectors.shape
    N = B * S
    info = plsc.get_sparse_core_info()
    n_workers = info.num_cores * info.num_subcores
    chunk = 400
    flat_idx = indices.reshape(N).astype(jnp.int32)
    out = _gather_kernel(N, V, D, n_workers, chunk)(flat_idx, vectors)
    return out.reshape(B, S, D)


# speedup vs baseline: 2.3262x; 1.2398x over previous
import functools

import jax
import jax.numpy as jnp
from jax import lax
from jax.experimental import pallas as pl
from jax.experimental.pallas import tpu as pltpu
from jax.experimental.pallas import tpu_sc as plsc

_NBUF = 4
_LOOKAHEAD = 2
_BPC = 4
_SEQ_PAD = 56


def _gather_kernel(n_batches, seq, V, n_workers):
    bat_per_w = n_batches // n_workers
    n_chunks = bat_per_w // _BPC
    n_groups = n_chunks // _NBUF
    chunk = _BPC * seq
    mesh = plsc.VectorSubcoreMesh(core_axis_name="c", subcore_axis_name="s")

    @functools.partial(
        pl.kernel,
        mesh=mesh,
        out_type=jax.ShapeDtypeStruct((n_batches * _SEQ_PAD, 128), jnp.float32),
        scratch_types=[
            pltpu.VMEM((_NBUF, chunk), jnp.int32),
            pltpu.VMEM((_NBUF, chunk, 128), jnp.float32),
            pltpu.SemaphoreType.DMA((_NBUF,)),
            pltpu.SemaphoreType.DMA((_NBUF,)),
        ],
        compiler_params=pltpu.CompilerParams(use_tc_tiling_on_sc=False),
    )
    def body(idx_hbm, table_hbm, out_hbm, idx_v, rows_v, g_sem, o_sem):
        nc = plsc.get_sparse_core_info().num_cores
        wid = lax.axis_index("s") * nc + lax.axis_index("c")
        bat0 = wid * bat_per_w

        def start_gather(i, slot):
            pltpu.sync_copy(
                idx_hbm.at[pl.ds((bat0 + i * _BPC) * seq, chunk)],
                idx_v.at[slot],
            )
            pltpu.async_copy(
                table_hbm.at[idx_v.at[slot]], rows_v.at[slot], g_sem.at[slot]
            )

        def wait_gather(i, slot):
            pltpu.make_async_copy(
                table_hbm.at[idx_v.at[slot]], rows_v.at[slot], g_sem.at[slot]
            ).wait()

        def batch_copies(i, slot):
            for k in range(_BPC):
                yield (
                    rows_v.at[slot].at[pl.ds(k * seq, seq)],
                    out_hbm.at[pl.ds((bat0 + i * _BPC + k) * _SEQ_PAD, seq)],
                )

        def start_writeout(i, slot):
            for src, dst in batch_copies(i, slot):
                pltpu.async_copy(src, dst, o_sem.at[slot])

        def wait_writeout(i, slot):
            for src, dst in batch_copies(i, slot):
                pltpu.make_async_copy(src, dst, o_sem.at[slot]).wait()

        for i in range(_LOOKAHEAD):
            start_gather(i, i % _NBUF)

        def group(g, carry):
            for b in range(_NBUF):
                i = g * _NBUF + b
                j_slot = (b + _LOOKAHEAD) % _NBUF
                prev = i + _LOOKAHEAD - _NBUF

                @pl.when(prev >= 0)
                def _():
                    wait_writeout(prev, j_slot)

                @pl.when(i + _LOOKAHEAD < n_chunks)
                def _():
                    start_gather(i + _LOOKAHEAD, j_slot)

                wait_gather(i, b)
                start_writeout(i, b)
            return carry

        lax.fori_loop(0, n_groups, group, 0)

        for k in range(_NBUF - _LOOKAHEAD):
            i = n_chunks - (_NBUF - _LOOKAHEAD) + k
            wait_writeout(i, i % _NBUF)

    return body


def kernel(indices, vectors):
    B, S = indices.shape
    V, D = vectors.shape
    info = plsc.get_sparse_core_info()
    n_workers = info.num_cores * info.num_subcores
    table_pad = jnp.pad(vectors, ((0, 0), (0, 128 - D)))
    flat_idx = indices.reshape(B * S).astype(jnp.int32)
    out2 = _gather_kernel(B, S, V, n_workers)(flat_idx, table_pad)
    return out2.reshape(B, _SEQ_PAD, 128)[:, :S, :D]
